# baseline (speedup 1.0000x reference)
import functools

import jax
import jax.numpy as jnp
from jax import lax
from jax.experimental import pallas as pl
from jax.experimental.pallas import tpu as pltpu
from jax.experimental.pallas import tpu_sc as plsc

_B = 1024
_L = 50
_V = 1000
_D = 1000
_NVT = 125
_NW = 32
_CHUNK = 8 * 8 * 128


def _body(idxT_hbm, tableT_hbm, out_hbm, idx_v, tT_v, chunk_v, *sems):
    wid = lax.axis_index("s") * 2 + lax.axis_index("c")

    pltpu.sync_copy(idxT_hbm, idx_v)
    for j in range(4):
        vt = wid + 32 * j

        @pl.when(vt < _NVT)
        def _stage():
            pltpu.sync_copy(tableT_hbm.at[pl.ds(vt * 8, 8)], tT_v.at[j])

    def start_scatter(l, vt, s):
        pltpu.make_async_copy(chunk_v.at[s], out_hbm.at[l, vt], sems[s]).start()

    def wait_scatter(s):
        pltpu.make_async_copy(chunk_v.at[s], out_hbm.at[0, 0], sems[s]).wait()

    def build_chunk(l, j, s):
        @plsc.parallel_loop(0, 64, unroll=16)
        def btk_body(i):
            bt = i >> 3
            k = i & 7
            ib = l * _B + bt * 128 + 16 * k
            cb = bt * 1024 + 16 * k
            x = idx_v[pl.ds(ib, 16)]
            for vs in range(8):
                val = plsc.load_gather(tT_v.at[j, vs], [x])
                chunk_v[s, pl.ds(cb + vs * 128, 16)] = val

    def unit(l, j, first_l):
        s = j % 2
        vt = wid + 32 * j

        @pl.when(vt < _NVT)
        def _run():
            if not (first_l and j < 2):
                if j == 1:
                    @pl.when(wid + 96 < _NVT)
                    def _w():
                        wait_scatter(s)
                else:
                    wait_scatter(s)
            build_chunk(l, j, s)
            start_scatter(l, vt, s)

    for j in range(4):
        unit(0, j, True)

    def l_body(l, carry):
        for j in range(4):
            unit(l, j, False)
        return carry

    lax.fori_loop(1, _L, l_body, 0)

    wait_scatter(0)
    wait_scatter(1)


_mesh = plsc.VectorSubcoreMesh(core_axis_name="c", subcore_axis_name="s")

_gather = functools.partial(
    pl.kernel,
    out_type=jax.ShapeDtypeStruct((_L, _NVT, _CHUNK), jnp.float32),
    mesh=_mesh,
    scratch_types=[
        pltpu.VMEM((_L * _B,), jnp.int32),
        pltpu.VMEM((4, 8, _D), jnp.float32),
        pltpu.VMEM((2, _CHUNK), jnp.float32),
        pltpu.SemaphoreType.DMA,
        pltpu.SemaphoreType.DMA,
    ],
    compiler_params=pltpu.CompilerParams(
        use_tc_tiling_on_sc=False, needs_layout_passes=False
    ),
)(_body)


@jax.jit
def kernel(idx, token_embedding_table):
    idx_t = idx.T.reshape(_L * _B)
    table_t = token_embedding_table.T
    out = _gather(idx_t, table_t)
    out5d = out.reshape(_L, _NVT, 8, 8, 128)
    return out5d.transpose(2, 4, 0, 1, 3).reshape(_B, _L, _V)


# speedup vs baseline: 1.0106x; 1.0106x over previous
import functools

import jax
import jax.numpy as jnp
from jax import lax
from jax.experimental import pallas as pl
from jax.experimental.pallas import tpu as pltpu
from jax.experimental.pallas import tpu_sc as plsc

_B = 1024
_L = 50
_V = 1000
_D = 1000
_NVT = 125
_NW = 32
_CHUNK = 8 * 8 * 128


def _body(idxT_hbm, tableT_hbm, out_hbm, idx_v, tT_v, chunk_v, *sems):
    wid = lax.axis_index("s") * 2 + lax.axis_index("c")

    pltpu.sync_copy(idxT_hbm, idx_v)
    for j in range(4):
        vt = wid + 32 * j

        @pl.when(vt < _NVT)
        def _stage():
            pltpu.sync_copy(tableT_hbm.at[pl.ds(vt * 8, 8)], tT_v.at[j])

    def start_scatter(l, vt, s):
        pltpu.make_async_copy(chunk_v.at[s], out_hbm.at[l, vt], sems[s]).start()

    def wait_scatter(s):
        pltpu.make_async_copy(chunk_v.at[s], out_hbm.at[0, 0], sems[s]).wait()

    def build_chunk(l, j, s):
        @plsc.parallel_loop(0, 64, unroll=8)
        def btk_body(i):
            bt = i >> 3
            k = i & 7
            ib = l * _B + bt * 128 + 16 * k
            cb = bt * 1024 + 16 * k
            x = idx_v[pl.ds(ib, 16)]
            for vs in range(8):
                val = plsc.load_gather(tT_v.at[j, vs], [x])
                chunk_v[s, pl.ds(cb + vs * 128, 16)] = val

    def unit(l, j, first_l):
        s = j % 2
        vt = wid + 32 * j

        @pl.when(vt < _NVT)
        def _run():
            if not (first_l and j < 2):
                if j == 1:
                    @pl.when(wid + 96 < _NVT)
                    def _w():
                        wait_scatter(s)
                else:
                    wait_scatter(s)
            build_chunk(l, j, s)
            start_scatter(l, vt, s)

    for j in range(4):
        unit(0, j, True)

    def l_body(l, carry):
        for j in range(4):
            unit(l, j, False)
        return carry

    lax.fori_loop(1, _L, l_body, 0)

    wait_scatter(0)
    wait_scatter(1)


_mesh = plsc.VectorSubcoreMesh(core_axis_name="c", subcore_axis_name="s")

_gather = functools.partial(
    pl.kernel,
    out_type=jax.ShapeDtypeStruct((_L, _NVT, _CHUNK), jnp.float32),
    mesh=_mesh,
    scratch_types=[
        pltpu.VMEM((_L * _B,), jnp.int32),
        pltpu.VMEM((4, 8, _D), jnp.float32),
        pltpu.VMEM((2, _CHUNK), jnp.float32),
        pltpu.SemaphoreType.DMA,
        pltpu.SemaphoreType.DMA,
    ],
    compiler_params=pltpu.CompilerParams(
        use_tc_tiling_on_sc=False, needs_layout_passes=False
    ),
)(_body)


@jax.jit
def kernel(idx, token_embedding_table):
    idx_t = idx.T.reshape(_L * _B)
    table_t = token_embedding_table.T
    out = _gather(idx_t, table_t)
    out5d = out.reshape(_L, _NVT, 8, 8, 128)
    return out5d.transpose(2, 4, 0, 1, 3).reshape(_B, _L, _V)
